# baseline (speedup 1.0000x reference)
import functools

import jax
import jax.numpy as jnp
from jax import lax
from jax.experimental import pallas as pl
from jax.experimental.pallas import tpu as pltpu
from jax.experimental.pallas import tpu_sc as plsc

NC = 2
NS = 16
NW = NC * NS
LANE = 16
EBLK = 128
CW = 128
LPAD = 128


def _ceil_to(a, m):
    return (a + m - 1) // m * m


DW = 128


def _deg_sc(dst2d, npad, eblk_rows_per_tile):
    epr = eblk_rows_per_tile
    stripe = npad // NS

    mesh = plsc.VectorSubcoreMesh(core_axis_name="c", subcore_axis_name="s")

    @functools.partial(
        pl.kernel,
        out_type=jax.ShapeDtypeStruct((NC, npad, DW), jnp.float32),
        mesh=mesh,
        scratch_types=[
            pltpu.VMEM_SHARED((npad, DW), jnp.float32),
            pltpu.VMEM((epr, EBLK), jnp.int32),
            pltpu.VMEM((EBLK, DW), jnp.float32),
        ],
    )
    def k(dst_hbm, ones_hbm, zero_hbm, out_hbm, slab, dst_v, ones_v):
        cid = lax.axis_index("c")
        sid = lax.axis_index("s")
        tid = cid * NS + sid

        pltpu.sync_copy(dst_hbm.at[pl.ds(tid * epr, epr)], dst_v)
        pltpu.sync_copy(ones_hbm, ones_v)
        pltpu.sync_copy(zero_hbm, slab.at[pl.ds(sid * stripe, stripe)])
        plsc.subcore_barrier()

        def edge_body(b, _):
            pltpu.sync_copy(ones_v, slab.at[dst_v.at[b]], add=True)
            return 0

        lax.fori_loop(0, epr, edge_body, 0)
        plsc.subcore_barrier()
        pltpu.sync_copy(
            slab.at[pl.ds(sid * stripe, stripe)],
            out_hbm.at[cid].at[pl.ds(sid * stripe, stripe)],
        )

    return k(dst2d, jnp.ones((EBLK, DW), jnp.float32),
             jnp.zeros((stripe, DW), jnp.float32))


def _prop_sc(ycg, src2d, dst2d, zeros_rows, nslab, r0, r1):
    cg = ycg.shape[0]
    stripe = nslab // NS

    mesh = plsc.VectorSubcoreMesh(core_axis_name="c", subcore_axis_name="s")

    NBUF = 2
    UNR = 8
    HALF = 40
    assert r0 % HALF == 0 and r1 % HALF == 0 and HALF % UNR == 0

    @functools.partial(
        pl.kernel,
        out_type=jax.ShapeDtypeStruct((NC, cg, nslab, CW), jnp.float32),
        mesh=mesh,
        scratch_types=(
            [pltpu.VMEM_SHARED((nslab, CW), jnp.float32)]
            + [pltpu.VMEM((HALF, EBLK), jnp.int32)] * 2
            + [pltpu.VMEM((EBLK, CW), jnp.float32)] * NBUF
            + [pltpu.SemaphoreType.DMA] * (2 * NBUF)
        ),
    )
    def k(y_hbm, src_hbm, dst_hbm, zero_hbm, out_hbm, slab, src_v, dst_v,
          *bufsems):
        rows = bufsems[:NBUF]
        semg = bufsems[NBUF:2 * NBUF]
        sems = bufsems[2 * NBUF:]
        cid = lax.axis_index("c")
        sid = lax.axis_index("s")
        rbase = jnp.where(cid == 0, sid * r0, NS * r0 + sid * r1)
        nchunk = jnp.where(cid == 0, r0 // HALF, r1 // HALF)

        def gather(b, i):
            pltpu.async_copy(y_hbm.at[g].at[src_v.at[b]], rows[i], semg[i])

        def scatter_start(b, i):
            pltpu.async_copy(rows[i], slab.at[dst_v.at[b]], sems[i], add=True)

        for g in range(cg):
            pltpu.sync_copy(zero_hbm, slab.at[pl.ds(sid * stripe, stripe)])
            plsc.subcore_barrier()

            def chunk_body(hh, _):
                base = rbase + hh * HALF
                pltpu.sync_copy(src_hbm.at[pl.ds(base, HALF)], src_v)
                pltpu.sync_copy(dst_hbm.at[pl.ds(base, HALF)], dst_v)
                for i in range(NBUF):
                    gather(i, i)

                def edge_body(j, _):
                    for k_ in range(UNR):
                        i = k_ % NBUF
                        b = j * UNR + k_
                        pltpu.make_async_copy(
                            y_hbm.at[g].at[src_v.at[b]], rows[i],
                            semg[i]).wait()
                        scatter_start(b, i)
                        pltpu.make_async_copy(
                            rows[i], slab.at[dst_v.at[b]], sems[i]).wait()

                        @pl.when(b + NBUF < HALF)
                        def _():
                            gather(b + NBUF, i)
                    return 0

                lax.fori_loop(0, HALF // UNR, edge_body, 0)
                return 0

            lax.fori_loop(0, nchunk, chunk_body, 0)
            plsc.subcore_barrier()
            pltpu.sync_copy(
                slab.at[pl.ds(sid * stripe, stripe)],
                out_hbm.at[cid, g].at[pl.ds(sid * stripe, stripe)],
            )

    return k(ycg, src2d, dst2d, zeros_rows)


_RB = 512


def _conv1_body(x_ref, w_ref, deg_ref, u0_ref, y_ref, disb_ref):
    acc = jnp.dot(x_ref[...], w_ref[...], preferred_element_type=jnp.float32)
    deg = deg_ref[0, :, 0] + deg_ref[1, :, 0]
    dis = jnp.where(deg > 0, lax.rsqrt(jnp.maximum(deg, 1e-12)), 0.0)
    h = acc.shape[1] // 2
    u0_ref[...] = acc[:, :h]
    disb = jnp.broadcast_to(dis[:, None], (acc.shape[0], CW))
    disb_ref[...] = disb
    for g in range(h // CW):
        y_ref[g] = disb * acc[:, h + g * CW:h + (g + 1) * CW]


def _conv1_tc(x, wcat, degpart, npad):
    h2 = wcat.shape[1]
    h = h2 // 2
    cg = h // CW
    grid = (npad // _RB,)
    return pl.pallas_call(
        _conv1_body,
        grid=grid,
        in_specs=[
            pl.BlockSpec((_RB, x.shape[1]), lambda i: (i, 0)),
            pl.BlockSpec(wcat.shape, lambda i: (0, 0)),
            pl.BlockSpec((NC, _RB, DW), lambda i: (0, i, 0)),
        ],
        out_specs=[
            pl.BlockSpec((_RB, h), lambda i: (i, 0)),
            pl.BlockSpec((cg, _RB, CW), lambda i: (0, i, 0)),
            pl.BlockSpec((_RB, CW), lambda i: (i, 0)),
        ],
        out_shape=[
            jax.ShapeDtypeStruct((npad, h), jnp.float32),
            jax.ShapeDtypeStruct((cg, npad, CW), jnp.float32),
            jax.ShapeDtypeStruct((npad, CW), jnp.float32),
        ],
    )(x, wcat, degpart)


def _combine_matmul_body(u0_ref, z_ref, disb_ref, b_ref, w_ref,
                         u0n_ref, y_ref):
    disb = disb_ref[...]
    cg = z_ref.shape[1]
    t = jnp.concatenate(
        [-disb * (z_ref[0, g] + z_ref[1, g]) for g in range(cg)], axis=1)
    hcur = jax.nn.relu(u0_ref[...] + t + b_ref[...])
    acc = jnp.dot(hcur, w_ref[...], preferred_element_type=jnp.float32)
    h = acc.shape[1] // 2
    u0n_ref[...] = acc[:, :h]
    for g in range(h // CW):
        y_ref[g] = disb * acc[:, h + g * CW:h + (g + 1) * CW]


def _combine_matmul_tc(u0, zpart, disb, bias, wcat, npad):
    n, h = u0.shape
    cg = h // CW
    grid = (n // _RB,)
    return pl.pallas_call(
        _combine_matmul_body,
        grid=grid,
        in_specs=[
            pl.BlockSpec((_RB, h), lambda i: (i, 0)),
            pl.BlockSpec((NC, cg, _RB, CW), lambda i: (0, 0, i, 0)),
            pl.BlockSpec((_RB, CW), lambda i: (i, 0)),
            pl.BlockSpec((1, h), lambda i: (0, 0)),
            pl.BlockSpec(wcat.shape, lambda i: (0, 0)),
        ],
        out_specs=[
            pl.BlockSpec((_RB, h), lambda i: (i, 0)),
            pl.BlockSpec((cg, _RB, CW), lambda i: (0, i, 0)),
        ],
        out_shape=[
            jax.ShapeDtypeStruct((n, h), jnp.float32),
            jax.ShapeDtypeStruct((cg, npad, CW), jnp.float32),
        ],
    )(u0, zpart, disb, bias, wcat)


def _combine_final_body(u0_ref, z_ref, disb_ref, b_ref, out_ref):
    disb = disb_ref[...]
    cg = z_ref.shape[1]
    t = jnp.concatenate(
        [-disb * (z_ref[0, g] + z_ref[1, g]) for g in range(cg)], axis=1)
    out_ref[...] = jax.nn.relu(u0_ref[...] + t + b_ref[...])


def _combine_final_tc(u0, zpart, disb, bias):
    n, h = u0.shape
    cg = h // CW
    grid = (n // _RB,)
    return pl.pallas_call(
        _combine_final_body,
        grid=grid,
        in_specs=[
            pl.BlockSpec((_RB, h), lambda i: (i, 0)),
            pl.BlockSpec((NC, cg, _RB, CW), lambda i: (0, 0, i, 0)),
            pl.BlockSpec((_RB, CW), lambda i: (i, 0)),
            pl.BlockSpec((1, h), lambda i: (0, 0)),
        ],
        out_specs=pl.BlockSpec((_RB, h), lambda i: (i, 0)),
        out_shape=jax.ShapeDtypeStruct((n, h), jnp.float32),
    )(u0, zpart, disb, bias)


def _pool_body(starts_ref, feat_ref, out_ref):
    g = pl.program_id(0)
    n, h = feat_ref.shape
    s = starts_ref[g]
    e = starts_ref[g + 1]
    s8 = pl.multiple_of((s // 8) * 8, 8)
    nch = (e - s8 + 7) // 8
    neg = jnp.full((8, h), -jnp.inf, jnp.float32)
    zero = jnp.zeros((8, h), jnp.float32)

    def body(j, carry):
        mx, sm = carry
        r0 = pl.multiple_of(s8 + j * 8, 8)
        rows = feat_ref[pl.ds(r0, 8), :]
        rid = r0 + lax.broadcasted_iota(jnp.int32, (8, h), 0)
        m = (rid >= s) & (rid < e)
        mx = jnp.maximum(mx, jnp.where(m, rows, -jnp.inf))
        sm = sm + jnp.where(m, rows, 0.0)
        return mx, sm

    mx, sm = lax.fori_loop(0, nch, body, (neg, zero))
    gmax = jnp.max(mx, axis=0)
    gmean = jnp.sum(sm, axis=0) / jnp.maximum((e - s).astype(jnp.float32), 1.0)
    out_ref[0, 0, :h] = gmax
    out_ref[0, 0, h:] = gmean


def _pool_tc(feature, starts, num_graphs):
    n, h = feature.shape
    grid_spec = pltpu.PrefetchScalarGridSpec(
        num_scalar_prefetch=1,
        grid=(num_graphs,),
        in_specs=[pl.BlockSpec((n, h), lambda g, starts: (0, 0))],
        out_specs=pl.BlockSpec((1, 1, 2 * h), lambda g, starts: (g, 0, 0)),
    )
    return pl.pallas_call(
        _pool_body,
        grid_spec=grid_spec,
        out_shape=jax.ShapeDtypeStruct((num_graphs, 1, 2 * h), jnp.float32),
    )(starts, feature).reshape(num_graphs, 2 * h)


def _heads_body(xg_ref, w01_ref, b01_ref, w02_ref, b02_ref, w03_ref, b03_ref,
                w11_ref, b11_ref, w12_ref, b12_ref, w13_ref, b13_ref,
                out0_ref, out1_ref, f1_ref):
    xg = xg_ref[...]

    def head(w1r, b1r, w2r, b2r, w3r, b3r, ncls):
        a0 = jax.nn.relu(
            jnp.dot(xg, w1r[...], preferred_element_type=jnp.float32)
            + b1r[...])
        a1 = jax.nn.relu(
            jnp.dot(a0, w2r[...], preferred_element_type=jnp.float32)
            + b2r[...])
        z = jax.nn.relu(
            jnp.dot(a1, w3r[...], preferred_element_type=jnp.float32)
            + b3r[...])
        mask = lax.broadcasted_iota(jnp.int32, z.shape, 1) < ncls
        zm = jnp.where(mask, z, -jnp.inf)
        m = jnp.max(zm, axis=1, keepdims=True)
        ex = jnp.where(mask, jnp.exp(zm - m), 0.0)
        return ex / jnp.sum(ex, axis=1, keepdims=True), a1

    out0, _ = head(w01_ref, b01_ref, w02_ref, b02_ref, w03_ref, b03_ref, 2)
    out1, a1 = head(w11_ref, b11_ref, w12_ref, b12_ref, w13_ref, b13_ref, 5)
    out0_ref[...] = out0
    out1_ref[...] = out1
    f1_ref[...] = a1


def _heads_tc(xg, ws):
    g = xg.shape[0]
    full = lambda a: pl.BlockSpec(a.shape, lambda: (0,) * a.ndim)
    return pl.pallas_call(
        _heads_body,
        in_specs=[full(xg)] + [full(w) for w in ws],
        out_specs=[
            pl.BlockSpec((g, LPAD), lambda: (0, 0)),
            pl.BlockSpec((g, LPAD), lambda: (0, 0)),
            pl.BlockSpec((g, ws[3].shape[1]), lambda: (0, 0)),
        ],
        out_shape=[
            jax.ShapeDtypeStruct((g, LPAD), jnp.float32),
            jax.ShapeDtypeStruct((g, LPAD), jnp.float32),
            jax.ShapeDtypeStruct((g, ws[3].shape[1]), jnp.float32),
        ],
    )(xg, *ws)


def _fold_bn(w, b, bn):
    g = bn['gamma'] * lax.rsqrt(bn['rv'] + 1e-5)
    return w * g[None, :], b * g + bn['beta'] - bn['rm'] * g


def _mix_branches(h, flag, feature_s):
    def ident(_):
        return h

    def mk(which):
        def f(_):
            prob = jax.random.normal(jax.random.key(1), (h.shape[0], 1),
                                     dtype=jnp.float32)
            miu = jnp.mean(h, axis=1, keepdims=True)
            sigma = jnp.std(h, axis=1, keepdims=True)
            if which == 0:
                s = sigma * (1 + prob)
                m = miu * (1 + prob)
            else:
                miu_s = jnp.mean(feature_s, axis=1, keepdims=True)
                sigma_s = jnp.std(feature_s, axis=1, keepdims=True)
                s = prob * sigma_s + (1 - prob) * sigma
                m = prob * miu_s + (1 - prob) * miu
            return s * (h - miu) / sigma + m
        return f

    flag = jnp.asarray(flag)
    idx = jnp.where(flag == 0, 0, jnp.where(flag == 1, 1, 2))
    return lax.switch(idx, [mk(0), mk(1), ident], 0)


def kernel(x, edge_index, batch, flag, feature_s, params):
    p = params
    n, _ = x.shape
    e = edge_index.shape[1]
    h = p['conv1_w0'].shape[1]
    num_graphs = 64

    npad = _ceil_to(n, NS * 8 * LANE)
    nslab = _ceil_to(n + 1, NS * 8)
    epad = _ceil_to(e, NW * EBLK)
    epr = epad // (NW * EBLK)
    r16 = epad // (NS * EBLK)
    r0 = r16
    r1 = 0
    dump = n

    src = jnp.concatenate(
        [edge_index[0], jnp.zeros((epad - e,), jnp.int32)]).reshape(-1, EBLK)
    dst = jnp.concatenate(
        [edge_index[1], jnp.full((epad - e,), dump, jnp.int32)]
    ).reshape(-1, EBLK)

    w10, b1f = _fold_bn(p['conv1_w0'], p['conv1_b'], p['bn1'])
    w11, _ = _fold_bn(p['conv1_w1'], p['conv1_b'], p['bn1'])
    w20, b2f = _fold_bn(p['conv2_w0'], p['conv2_b'], p['bn2'])
    w21, _ = _fold_bn(p['conv2_w1'], p['conv2_b'], p['bn2'])
    w1cat = jnp.concatenate([w10, w11], axis=1)
    w2cat = jnp.concatenate([w20, w21], axis=1)

    degpart = _deg_sc(dst, npad, epr)
    u0, ycg, disb = _conv1_tc(x, w1cat, degpart, npad)

    zeros_rows = jnp.zeros((nslab // NS, CW), jnp.float32)
    zpart1 = _prop_sc(ycg, src, dst, zeros_rows, nslab, r0, r1)
    u0b, ycg2 = _combine_matmul_tc(u0, zpart1, disb, b1f[None, :], w2cat,
                                   npad)
    zpart2 = _prop_sc(ycg2, src, dst, zeros_rows, nslab, r0, r1)
    feature = _combine_final_tc(u0b, zpart2, disb, b2f[None, :])[:n]

    feature = _mix_branches(feature, flag, feature_s)

    starts = jnp.searchsorted(
        batch, jnp.arange(num_graphs + 1, dtype=jnp.int32)).astype(jnp.int32)
    xg = _pool_tc(feature, starts, num_graphs)

    wl01, bl01 = _fold_bn(p['lin01_w'], p['lin01_b'], p['bn01'])
    wl02, bl02 = _fold_bn(p['lin02_w'], p['lin02_b'], p['bn02'])
    wl11, bl11 = _fold_bn(p['lin11_w'], p['lin11_b'], p['bn11'])
    wl12, bl12 = _fold_bn(p['lin12_w'], p['lin12_b'], p['bn12'])
    pad_w = lambda w: jnp.pad(w, ((0, 0), (0, LPAD - w.shape[1])))
    pad_b = lambda b: jnp.pad(b, (0, LPAD - b.shape[0]))
    ws = [
        wl01, bl01[None, :], wl02, bl02[None, :],
        pad_w(p['lin03_w']), pad_b(p['lin03_b'])[None, :],
        wl11, bl11[None, :], wl12, bl12[None, :],
        pad_w(p['lin13_w']), pad_b(p['lin13_b'])[None, :],
    ]
    out0p, out1p, feature1 = _heads_tc(xg, ws)
    n0 = p['lin03_w'].shape[1]
    n1 = p['lin13_w'].shape[1]
    return (out0p[:, :n0], out1p[:, :n1], feature, xg, feature1)


# speedup vs baseline: 1.0564x; 1.0564x over previous
import functools

import jax
import jax.numpy as jnp
from jax import lax
from jax.experimental import pallas as pl
from jax.experimental.pallas import tpu as pltpu
from jax.experimental.pallas import tpu_sc as plsc

NC = 2
NS = 16
NW = NC * NS
LANE = 16
EBLK = 128
CW = 128


def _ceil_to(a, m):
    return (a + m - 1) // m * m


DW = 128


def _deg_sc(dst2d, npad, eblk_rows_per_tile):
    epr = eblk_rows_per_tile
    stripe = npad // NS

    mesh = plsc.VectorSubcoreMesh(core_axis_name="c", subcore_axis_name="s")

    @functools.partial(
        pl.kernel,
        out_type=jax.ShapeDtypeStruct((NC, npad, DW), jnp.float32),
        mesh=mesh,
        scratch_types=[
            pltpu.VMEM_SHARED((npad, DW), jnp.float32),
            pltpu.VMEM((epr, EBLK), jnp.int32),
            pltpu.VMEM((EBLK, DW), jnp.float32),
        ],
    )
    def k(dst_hbm, ones_hbm, zero_hbm, out_hbm, slab, dst_v, ones_v):
        cid = lax.axis_index("c")
        sid = lax.axis_index("s")
        tid = cid * NS + sid

        pltpu.sync_copy(dst_hbm.at[pl.ds(tid * epr, epr)], dst_v)
        pltpu.sync_copy(ones_hbm, ones_v)
        pltpu.sync_copy(zero_hbm, slab.at[pl.ds(sid * stripe, stripe)])
        plsc.subcore_barrier()

        def edge_body(b, _):
            pltpu.sync_copy(ones_v, slab.at[dst_v.at[b]], add=True)
            return 0

        lax.fori_loop(0, epr, edge_body, 0)
        plsc.subcore_barrier()
        pltpu.sync_copy(
            slab.at[pl.ds(sid * stripe, stripe)],
            out_hbm.at[cid].at[pl.ds(sid * stripe, stripe)],
        )

    return k(dst2d, jnp.ones((EBLK, DW), jnp.float32),
             jnp.zeros((stripe, DW), jnp.float32))


def _prop_sc(ycg, src2d, dst2d, zeros_rows, npad, eblk_rows_per_tile):
    cg = ycg.shape[0]
    epr = eblk_rows_per_tile
    stripe = npad // NS

    mesh = plsc.VectorSubcoreMesh(core_axis_name="c", subcore_axis_name="s")

    NBUF = 2
    UNR = 8
    assert epr % UNR == 0

    @functools.partial(
        pl.kernel,
        out_type=jax.ShapeDtypeStruct((NC, cg, npad, CW), jnp.float32),
        mesh=mesh,
        scratch_types=(
            [pltpu.VMEM_SHARED((npad, CW), jnp.float32)]
            + [pltpu.VMEM((epr, EBLK), jnp.int32)] * 2
            + [pltpu.VMEM((EBLK, CW), jnp.float32)] * NBUF
            + [pltpu.SemaphoreType.DMA] * (2 * NBUF)
        ),
    )
    def k(y_hbm, src_hbm, dst_hbm, zero_hbm, out_hbm, slab, src_v, dst_v,
          *bufsems):
        rows = bufsems[:NBUF]
        semg = bufsems[NBUF:2 * NBUF]
        sems = bufsems[2 * NBUF:]
        cid = lax.axis_index("c")
        sid = lax.axis_index("s")
        tid = cid * NS + sid

        pltpu.sync_copy(src_hbm.at[pl.ds(tid * epr, epr)], src_v)
        pltpu.sync_copy(dst_hbm.at[pl.ds(tid * epr, epr)], dst_v)

        def gather(b, i):
            pltpu.async_copy(y_hbm.at[g].at[src_v.at[b]], rows[i], semg[i])

        def scatter_start(b, i):
            pltpu.async_copy(rows[i], slab.at[dst_v.at[b]], sems[i], add=True)

        for g in range(cg):
            pltpu.sync_copy(zero_hbm, slab.at[pl.ds(sid * stripe, stripe)])
            plsc.subcore_barrier()

            for i in range(NBUF):
                gather(i, i)

            def edge_body(j, _):
                for k_ in range(UNR):
                    i = k_ % NBUF
                    b = j * UNR + k_
                    pltpu.make_async_copy(
                        y_hbm.at[g].at[src_v.at[b]], rows[i], semg[i]).wait()
                    scatter_start(b, i)
                    pltpu.make_async_copy(
                        rows[i], slab.at[dst_v.at[b]], sems[i]).wait()

                    @pl.when(b + NBUF < epr)
                    def _():
                        gather(b + NBUF, i)
                return 0

            lax.fori_loop(0, epr // UNR, edge_body, 0)
            plsc.subcore_barrier()
            pltpu.sync_copy(
                slab.at[pl.ds(sid * stripe, stripe)],
                out_hbm.at[cid, g].at[pl.ds(sid * stripe, stripe)],
            )

    return k(ycg, src2d, dst2d, zeros_rows)


_RB = 512


def _conv1_body(x_ref, w_ref, deg_ref, u0_ref, y_ref, disb_ref):
    acc = jnp.dot(x_ref[...], w_ref[...], preferred_element_type=jnp.float32)
    deg = deg_ref[0, :, 0] + deg_ref[1, :, 0]
    dis = jnp.where(deg > 0, lax.rsqrt(jnp.maximum(deg, 1e-12)), 0.0)
    h = acc.shape[1] // 2
    u0_ref[...] = acc[:, :h]
    disb = jnp.broadcast_to(dis[:, None], (acc.shape[0], CW))
    disb_ref[...] = disb
    for g in range(h // CW):
        y_ref[g] = disb * acc[:, h + g * CW:h + (g + 1) * CW]


def _conv1_tc(x, wcat, degpart, npad):
    h2 = wcat.shape[1]
    h = h2 // 2
    cg = h // CW
    grid = (npad // _RB,)
    return pl.pallas_call(
        _conv1_body,
        grid=grid,
        in_specs=[
            pl.BlockSpec((_RB, x.shape[1]), lambda i: (i, 0)),
            pl.BlockSpec(wcat.shape, lambda i: (0, 0)),
            pl.BlockSpec((NC, _RB, DW), lambda i: (0, i, 0)),
        ],
        out_specs=[
            pl.BlockSpec((_RB, h), lambda i: (i, 0)),
            pl.BlockSpec((cg, _RB, CW), lambda i: (0, i, 0)),
            pl.BlockSpec((_RB, CW), lambda i: (i, 0)),
        ],
        out_shape=[
            jax.ShapeDtypeStruct((npad, h), jnp.float32),
            jax.ShapeDtypeStruct((cg, npad, CW), jnp.float32),
            jax.ShapeDtypeStruct((npad, CW), jnp.float32),
        ],
    )(x, wcat, degpart)


def _combine_matmul_body(u0_ref, z_ref, disb_ref, b_ref, w_ref,
                         u0n_ref, y_ref):
    disb = disb_ref[...]
    cg = z_ref.shape[1]
    t = jnp.concatenate(
        [-disb * (z_ref[0, g] + z_ref[1, g]) for g in range(cg)], axis=1)
    hcur = jax.nn.relu(u0_ref[...] + t + b_ref[...])
    acc = jnp.dot(hcur, w_ref[...], preferred_element_type=jnp.float32)
    h = acc.shape[1] // 2
    u0n_ref[...] = acc[:, :h]
    for g in range(h // CW):
        y_ref[g] = disb * acc[:, h + g * CW:h + (g + 1) * CW]


def _combine_matmul_tc(u0, zpart, disb, bias, wcat, npad):
    n, h = u0.shape
    cg = h // CW
    grid = (n // _RB,)
    return pl.pallas_call(
        _combine_matmul_body,
        grid=grid,
        in_specs=[
            pl.BlockSpec((_RB, h), lambda i: (i, 0)),
            pl.BlockSpec((NC, cg, _RB, CW), lambda i: (0, 0, i, 0)),
            pl.BlockSpec((_RB, CW), lambda i: (i, 0)),
            pl.BlockSpec((1, h), lambda i: (0, 0)),
            pl.BlockSpec(wcat.shape, lambda i: (0, 0)),
        ],
        out_specs=[
            pl.BlockSpec((_RB, h), lambda i: (i, 0)),
            pl.BlockSpec((cg, _RB, CW), lambda i: (0, i, 0)),
        ],
        out_shape=[
            jax.ShapeDtypeStruct((n, h), jnp.float32),
            jax.ShapeDtypeStruct((cg, npad, CW), jnp.float32),
        ],
    )(u0, zpart, disb, bias, wcat)


def _combine_final_body(u0_ref, z_ref, disb_ref, b_ref, out_ref):
    disb = disb_ref[...]
    cg = z_ref.shape[1]
    t = jnp.concatenate(
        [-disb * (z_ref[0, g] + z_ref[1, g]) for g in range(cg)], axis=1)
    out_ref[...] = jax.nn.relu(u0_ref[...] + t + b_ref[...])


def _combine_final_tc(u0, zpart, disb, bias):
    n, h = u0.shape
    cg = h // CW
    grid = (n // _RB,)
    return pl.pallas_call(
        _combine_final_body,
        grid=grid,
        in_specs=[
            pl.BlockSpec((_RB, h), lambda i: (i, 0)),
            pl.BlockSpec((NC, cg, _RB, CW), lambda i: (0, 0, i, 0)),
            pl.BlockSpec((_RB, CW), lambda i: (i, 0)),
            pl.BlockSpec((1, h), lambda i: (0, 0)),
        ],
        out_specs=pl.BlockSpec((_RB, h), lambda i: (i, 0)),
        out_shape=jax.ShapeDtypeStruct((n, h), jnp.float32),
    )(u0, zpart, disb, bias)


def _pool_body(starts_ref, feat_ref, out_ref):
    g = pl.program_id(0)
    n, h = feat_ref.shape
    s = starts_ref[g]
    e = starts_ref[g + 1]
    s8 = pl.multiple_of((s // 8) * 8, 8)
    nch = (e - s8 + 7) // 8
    neg = jnp.full((8, h), -jnp.inf, jnp.float32)
    zero = jnp.zeros((8, h), jnp.float32)

    def body(j, carry):
        mx, sm = carry
        r0 = pl.multiple_of(s8 + j * 8, 8)
        rows = feat_ref[pl.ds(r0, 8), :]
        rid = r0 + lax.broadcasted_iota(jnp.int32, (8, h), 0)
        m = (rid >= s) & (rid < e)
        mx = jnp.maximum(mx, jnp.where(m, rows, -jnp.inf))
        sm = sm + jnp.where(m, rows, 0.0)
        return mx, sm

    mx, sm = lax.fori_loop(0, nch, body, (neg, zero))
    gmax = jnp.max(mx, axis=0)
    gmean = jnp.sum(sm, axis=0) / jnp.maximum((e - s).astype(jnp.float32), 1.0)
    out_ref[0, 0, :h] = gmax
    out_ref[0, 0, h:] = gmean


def _pool_tc(feature, starts, num_graphs):
    n, h = feature.shape
    grid_spec = pltpu.PrefetchScalarGridSpec(
        num_scalar_prefetch=1,
        grid=(num_graphs,),
        in_specs=[pl.BlockSpec((n, h), lambda g, starts: (0, 0))],
        out_specs=pl.BlockSpec((1, 1, 2 * h), lambda g, starts: (g, 0, 0)),
    )
    return pl.pallas_call(
        _pool_body,
        grid_spec=grid_spec,
        out_shape=jax.ShapeDtypeStruct((num_graphs, 1, 2 * h), jnp.float32),
    )(starts, feature).reshape(num_graphs, 2 * h)


def _heads_body(xg_ref, w01_ref, b01_ref, w02_ref, b02_ref, w03_ref, b03_ref,
                w11_ref, b11_ref, w12_ref, b12_ref, w13_ref, b13_ref,
                out0_ref, out1_ref, f1_ref):
    xg = xg_ref[...]

    def head(w1r, b1r, w2r, b2r, w3r, b3r, ncls):
        a0 = jax.nn.relu(
            jnp.dot(xg, w1r[...], preferred_element_type=jnp.float32)
            + b1r[...])
        a1 = jax.nn.relu(
            jnp.dot(a0, w2r[...], preferred_element_type=jnp.float32)
            + b2r[...])
        z = jax.nn.relu(
            jnp.dot(a1, w3r[...], preferred_element_type=jnp.float32)
            + b3r[...])
        mask = lax.broadcasted_iota(jnp.int32, z.shape, 1) < ncls
        zm = jnp.where(mask, z, -jnp.inf)
        m = jnp.max(zm, axis=1, keepdims=True)
        ex = jnp.where(mask, jnp.exp(zm - m), 0.0)
        return ex / jnp.sum(ex, axis=1, keepdims=True), a1

    out0, _ = head(w01_ref, b01_ref, w02_ref, b02_ref, w03_ref, b03_ref, 2)
    out1, a1 = head(w11_ref, b11_ref, w12_ref, b12_ref, w13_ref, b13_ref, 5)
    out0_ref[...] = out0
    out1_ref[...] = out1
    f1_ref[...] = a1


def _heads_tc(xg, ws):
    g = xg.shape[0]
    full = lambda a: pl.BlockSpec(a.shape, lambda: (0,) * a.ndim)
    return pl.pallas_call(
        _heads_body,
        in_specs=[full(xg)] + [full(w) for w in ws],
        out_specs=[
            pl.BlockSpec((g, CW), lambda: (0, 0)),
            pl.BlockSpec((g, CW), lambda: (0, 0)),
            pl.BlockSpec((g, ws[3].shape[1]), lambda: (0, 0)),
        ],
        out_shape=[
            jax.ShapeDtypeStruct((g, CW), jnp.float32),
            jax.ShapeDtypeStruct((g, CW), jnp.float32),
            jax.ShapeDtypeStruct((g, ws[3].shape[1]), jnp.float32),
        ],
    )(xg, *ws)


def _fold_bn(w, b, bn):
    g = bn['gamma'] * lax.rsqrt(bn['rv'] + 1e-5)
    return w * g[None, :], b * g + bn['beta'] - bn['rm'] * g


def _mix_branches(h, flag, feature_s):
    def ident(_):
        return h

    def mk(which):
        def f(_):
            prob = jax.random.normal(jax.random.key(1), (h.shape[0], 1),
                                     dtype=jnp.float32)
            miu = jnp.mean(h, axis=1, keepdims=True)
            sigma = jnp.std(h, axis=1, keepdims=True)
            if which == 0:
                s = sigma * (1 + prob)
                m = miu * (1 + prob)
            else:
                miu_s = jnp.mean(feature_s, axis=1, keepdims=True)
                sigma_s = jnp.std(feature_s, axis=1, keepdims=True)
                s = prob * sigma_s + (1 - prob) * sigma
                m = prob * miu_s + (1 - prob) * miu
            return s * (h - miu) / sigma + m
        return f

    flag = jnp.asarray(flag)
    idx = jnp.where(flag == 0, 0, jnp.where(flag == 1, 1, 2))
    return lax.switch(idx, [mk(0), mk(1), ident], 0)


def kernel(x, edge_index, batch, flag, feature_s, params):
    p = params
    n, _ = x.shape
    e = edge_index.shape[1]
    h = p['conv1_w0'].shape[1]
    num_graphs = 64

    npad = _ceil_to(n, NS * 8 * LANE)
    epad = _ceil_to(e, NW * EBLK)
    epr = epad // (NW * EBLK)
    dump = n

    skey = jnp.sort(edge_index[0] * 16384 + edge_index[1])
    src = jnp.concatenate(
        [skey >> 14, jnp.zeros((epad - e,), jnp.int32)]).reshape(-1, EBLK)
    dst = jnp.concatenate(
        [skey & 16383, jnp.full((epad - e,), dump, jnp.int32)]
    ).reshape(-1, EBLK)

    w10, b1f = _fold_bn(p['conv1_w0'], p['conv1_b'], p['bn1'])
    w11, _ = _fold_bn(p['conv1_w1'], p['conv1_b'], p['bn1'])
    w20, b2f = _fold_bn(p['conv2_w0'], p['conv2_b'], p['bn2'])
    w21, _ = _fold_bn(p['conv2_w1'], p['conv2_b'], p['bn2'])
    w1cat = jnp.concatenate([w10, w11], axis=1)
    w2cat = jnp.concatenate([w20, w21], axis=1)

    degpart = _deg_sc(dst, npad, epr)
    u0, ycg, disb = _conv1_tc(x, w1cat, degpart, npad)

    zeros_rows = jnp.zeros((npad // NS, CW), jnp.float32)
    zpart1 = _prop_sc(ycg, src, dst, zeros_rows, npad, epr)
    u0b, ycg2 = _combine_matmul_tc(u0, zpart1, disb, b1f[None, :], w2cat,
                                   npad)
    zpart2 = _prop_sc(ycg2, src, dst, zeros_rows, npad, epr)
    feature = _combine_final_tc(u0b, zpart2, disb, b2f[None, :])[:n]

    feature = _mix_branches(feature, flag, feature_s)

    starts = jnp.searchsorted(
        batch, jnp.arange(num_graphs + 1, dtype=jnp.int32)).astype(jnp.int32)
    xg = _pool_tc(feature, starts, num_graphs)

    wl01, bl01 = _fold_bn(p['lin01_w'], p['lin01_b'], p['bn01'])
    wl02, bl02 = _fold_bn(p['lin02_w'], p['lin02_b'], p['bn02'])
    wl11, bl11 = _fold_bn(p['lin11_w'], p['lin11_b'], p['bn11'])
    wl12, bl12 = _fold_bn(p['lin12_w'], p['lin12_b'], p['bn12'])
    pad_w = lambda w: jnp.pad(w, ((0, 0), (0, CW - w.shape[1])))
    pad_b = lambda b: jnp.pad(b, (0, CW - b.shape[0]))
    ws = [
        wl01, bl01[None, :], wl02, bl02[None, :],
        pad_w(p['lin03_w']), pad_b(p['lin03_b'])[None, :],
        wl11, bl11[None, :], wl12, bl12[None, :],
        pad_w(p['lin13_w']), pad_b(p['lin13_b'])[None, :],
    ]
    out0p, out1p, feature1 = _heads_tc(xg, ws)
    n0 = p['lin03_w'].shape[1]
    n1 = p['lin13_w'].shape[1]
    return (out0p[:, :n0], out1p[:, :n1], feature, xg, feature1)


# speedup vs baseline: 1.2470x; 1.1803x over previous
import functools

import jax
import jax.numpy as jnp
from jax import lax
from jax.experimental import pallas as pl
from jax.experimental.pallas import tpu as pltpu
from jax.experimental.pallas import tpu_sc as plsc

NC = 2
NS = 16
NW = NC * NS
LANE = 16
EBLK = 128
CW = 128


def _ceil_to(a, m):
    return (a + m - 1) // m * m


DW = 128


def _deg_sc(dst2d, npad, eblk_rows_per_tile):
    epr = eblk_rows_per_tile
    stripe = npad // NS

    mesh = plsc.VectorSubcoreMesh(core_axis_name="c", subcore_axis_name="s")

    @functools.partial(
        pl.kernel,
        out_type=jax.ShapeDtypeStruct((NC, npad, DW), jnp.float32),
        mesh=mesh,
        scratch_types=[
            pltpu.VMEM_SHARED((npad, DW), jnp.float32),
            pltpu.VMEM((epr, EBLK), jnp.int32),
            pltpu.VMEM((EBLK, DW), jnp.float32),
        ],
    )
    def k(dst_hbm, ones_hbm, zero_hbm, out_hbm, slab, dst_v, ones_v):
        cid = lax.axis_index("c")
        sid = lax.axis_index("s")
        tid = cid * NS + sid

        pltpu.sync_copy(dst_hbm.at[pl.ds(tid * epr, epr)], dst_v)
        pltpu.sync_copy(ones_hbm, ones_v)
        pltpu.sync_copy(zero_hbm, slab.at[pl.ds(sid * stripe, stripe)])
        plsc.subcore_barrier()

        def edge_body(b, _):
            pltpu.sync_copy(ones_v, slab.at[dst_v.at[b]], add=True)
            return 0

        lax.fori_loop(0, epr, edge_body, 0)
        plsc.subcore_barrier()
        pltpu.sync_copy(
            slab.at[pl.ds(sid * stripe, stripe)],
            out_hbm.at[cid].at[pl.ds(sid * stripe, stripe)],
        )

    return k(dst2d, jnp.ones((EBLK, DW), jnp.float32),
             jnp.zeros((stripe, DW), jnp.float32))


def _prop_sc(ycg, src2d, dst2d, zeros_rows, npad, eblk_rows_per_tile):
    cg = ycg.shape[0]
    epr = eblk_rows_per_tile
    stripe = npad // NS

    mesh = plsc.VectorSubcoreMesh(core_axis_name="c", subcore_axis_name="s")

    NBUF = 2
    UNR = 8
    assert epr % UNR == 0

    @functools.partial(
        pl.kernel,
        out_type=jax.ShapeDtypeStruct((NC, cg, npad, CW), jnp.float32),
        mesh=mesh,
        scratch_types=(
            [pltpu.VMEM_SHARED((npad, CW), jnp.float32)]
            + [pltpu.VMEM((epr, EBLK), jnp.int32)] * 2
            + [pltpu.VMEM((EBLK, CW), jnp.float32)] * NBUF
            + [pltpu.SemaphoreType.DMA] * (2 * NBUF)
        ),
    )
    def k(y_hbm, src_hbm, dst_hbm, zero_hbm, out_hbm, slab, src_v, dst_v,
          *bufsems):
        rows = bufsems[:NBUF]
        semg = bufsems[NBUF:2 * NBUF]
        sems = bufsems[2 * NBUF:]
        cid = lax.axis_index("c")
        sid = lax.axis_index("s")
        tid = cid * NS + sid

        pltpu.sync_copy(src_hbm.at[pl.ds(tid * epr, epr)], src_v)
        pltpu.sync_copy(dst_hbm.at[pl.ds(tid * epr, epr)], dst_v)

        def gather(b, i):
            pltpu.async_copy(y_hbm.at[g].at[src_v.at[b]], rows[i], semg[i])

        def scatter_start(b, i):
            pltpu.async_copy(rows[i], slab.at[dst_v.at[b]], sems[i], add=True)

        for g in range(cg):
            pltpu.sync_copy(zero_hbm, slab.at[pl.ds(sid * stripe, stripe)])
            plsc.subcore_barrier()

            for i in range(NBUF):
                gather(i, i)

            def edge_body(j, _):
                for k_ in range(UNR):
                    i = k_ % NBUF
                    b = j * UNR + k_
                    pltpu.make_async_copy(
                        y_hbm.at[g].at[src_v.at[b]], rows[i], semg[i]).wait()
                    scatter_start(b, i)
                    pltpu.make_async_copy(
                        rows[i], slab.at[dst_v.at[b]], sems[i]).wait()

                    @pl.when(b + NBUF < epr)
                    def _():
                        gather(b + NBUF, i)
                return 0

            lax.fori_loop(0, epr // UNR, edge_body, 0)
            plsc.subcore_barrier()
            pltpu.sync_copy(
                slab.at[pl.ds(sid * stripe, stripe)],
                out_hbm.at[cid, g].at[pl.ds(sid * stripe, stripe)],
            )

    return k(ycg, src2d, dst2d, zeros_rows)


_RB = 512


def _conv1_body(x_ref, w_ref, deg_ref, u0_ref, y_ref, disb_ref):
    acc = jnp.dot(x_ref[...], w_ref[...], preferred_element_type=jnp.float32)
    deg = deg_ref[0, :, 0] + deg_ref[1, :, 0]
    dis = jnp.where(deg > 0, lax.rsqrt(jnp.maximum(deg, 1e-12)), 0.0)
    h = acc.shape[1] // 2
    u0_ref[...] = acc[:, :h]
    disb = jnp.broadcast_to(dis[:, None], (acc.shape[0], CW))
    disb_ref[...] = disb
    for g in range(h // CW):
        y_ref[g] = disb * acc[:, h + g * CW:h + (g + 1) * CW]


def _conv1_tc(x, wcat, degpart, npad):
    h2 = wcat.shape[1]
    h = h2 // 2
    cg = h // CW
    grid = (npad // _RB,)
    return pl.pallas_call(
        _conv1_body,
        grid=grid,
        in_specs=[
            pl.BlockSpec((_RB, x.shape[1]), lambda i: (i, 0)),
            pl.BlockSpec(wcat.shape, lambda i: (0, 0)),
            pl.BlockSpec((NC, _RB, DW), lambda i: (0, i, 0)),
        ],
        out_specs=[
            pl.BlockSpec((_RB, h), lambda i: (i, 0)),
            pl.BlockSpec((cg, _RB, CW), lambda i: (0, i, 0)),
            pl.BlockSpec((_RB, CW), lambda i: (i, 0)),
        ],
        out_shape=[
            jax.ShapeDtypeStruct((npad, h), jnp.float32),
            jax.ShapeDtypeStruct((cg, npad, CW), jnp.float32),
            jax.ShapeDtypeStruct((npad, CW), jnp.float32),
        ],
    )(x, wcat, degpart)


def _combine_matmul_body(u0_ref, z_ref, disb_ref, b_ref, w_ref,
                         u0n_ref, y_ref):
    disb = disb_ref[...]
    cg = z_ref.shape[1]
    t = jnp.concatenate(
        [-disb * (z_ref[0, g] + z_ref[1, g]) for g in range(cg)], axis=1)
    hcur = jax.nn.relu(u0_ref[...] + t + b_ref[...])
    acc = jnp.dot(hcur, w_ref[...], preferred_element_type=jnp.float32)
    h = acc.shape[1] // 2
    u0n_ref[...] = acc[:, :h]
    for g in range(h // CW):
        y_ref[g] = disb * acc[:, h + g * CW:h + (g + 1) * CW]


def _combine_matmul_tc(u0, zpart, disb, bias, wcat, npad):
    n, h = u0.shape
    cg = h // CW
    grid = (n // _RB,)
    return pl.pallas_call(
        _combine_matmul_body,
        grid=grid,
        in_specs=[
            pl.BlockSpec((_RB, h), lambda i: (i, 0)),
            pl.BlockSpec((NC, cg, _RB, CW), lambda i: (0, 0, i, 0)),
            pl.BlockSpec((_RB, CW), lambda i: (i, 0)),
            pl.BlockSpec((1, h), lambda i: (0, 0)),
            pl.BlockSpec(wcat.shape, lambda i: (0, 0)),
        ],
        out_specs=[
            pl.BlockSpec((_RB, h), lambda i: (i, 0)),
            pl.BlockSpec((cg, _RB, CW), lambda i: (0, i, 0)),
        ],
        out_shape=[
            jax.ShapeDtypeStruct((n, h), jnp.float32),
            jax.ShapeDtypeStruct((cg, npad, CW), jnp.float32),
        ],
    )(u0, zpart, disb, bias, wcat)


def _combine_final_body(u0_ref, z_ref, disb_ref, b_ref, out_ref):
    disb = disb_ref[...]
    cg = z_ref.shape[1]
    t = jnp.concatenate(
        [-disb * (z_ref[0, g] + z_ref[1, g]) for g in range(cg)], axis=1)
    out_ref[...] = jax.nn.relu(u0_ref[...] + t + b_ref[...])


def _combine_final_tc(u0, zpart, disb, bias):
    n, h = u0.shape
    cg = h // CW
    grid = (n // _RB,)
    return pl.pallas_call(
        _combine_final_body,
        grid=grid,
        in_specs=[
            pl.BlockSpec((_RB, h), lambda i: (i, 0)),
            pl.BlockSpec((NC, cg, _RB, CW), lambda i: (0, 0, i, 0)),
            pl.BlockSpec((_RB, CW), lambda i: (i, 0)),
            pl.BlockSpec((1, h), lambda i: (0, 0)),
        ],
        out_specs=pl.BlockSpec((_RB, h), lambda i: (i, 0)),
        out_shape=jax.ShapeDtypeStruct((n, h), jnp.float32),
    )(u0, zpart, disb, bias)


def _pool_body(starts_ref, feat_ref, out_ref):
    g = pl.program_id(0)
    n, h = feat_ref.shape
    s = starts_ref[g]
    e = starts_ref[g + 1]
    s8 = pl.multiple_of((s // 8) * 8, 8)
    nch = (e - s8 + 7) // 8
    neg = jnp.full((8, h), -jnp.inf, jnp.float32)
    zero = jnp.zeros((8, h), jnp.float32)

    def body(j, carry):
        mx, sm = carry
        r0 = pl.multiple_of(s8 + j * 8, 8)
        rows = feat_ref[pl.ds(r0, 8), :]
        rid = r0 + lax.broadcasted_iota(jnp.int32, (8, h), 0)
        m = (rid >= s) & (rid < e)
        mx = jnp.maximum(mx, jnp.where(m, rows, -jnp.inf))
        sm = sm + jnp.where(m, rows, 0.0)
        return mx, sm

    mx, sm = lax.fori_loop(0, nch, body, (neg, zero))
    gmax = jnp.max(mx, axis=0)
    gmean = jnp.sum(sm, axis=0) / jnp.maximum((e - s).astype(jnp.float32), 1.0)
    out_ref[0, 0, :h] = gmax
    out_ref[0, 0, h:] = gmean


def _pool_tc(feature, starts, num_graphs):
    n, h = feature.shape
    grid_spec = pltpu.PrefetchScalarGridSpec(
        num_scalar_prefetch=1,
        grid=(num_graphs,),
        in_specs=[pl.BlockSpec((n, h), lambda g, starts: (0, 0))],
        out_specs=pl.BlockSpec((1, 1, 2 * h), lambda g, starts: (g, 0, 0)),
    )
    return pl.pallas_call(
        _pool_body,
        grid_spec=grid_spec,
        out_shape=jax.ShapeDtypeStruct((num_graphs, 1, 2 * h), jnp.float32),
    )(starts, feature).reshape(num_graphs, 2 * h)


def _heads_body(xg_ref, w01_ref, b01_ref, w02_ref, b02_ref, w03_ref, b03_ref,
                w11_ref, b11_ref, w12_ref, b12_ref, w13_ref, b13_ref,
                out0_ref, out1_ref, f1_ref):
    xg = xg_ref[...]

    def head(w1r, b1r, w2r, b2r, w3r, b3r, ncls):
        a0 = jax.nn.relu(
            jnp.dot(xg, w1r[...], preferred_element_type=jnp.float32)
            + b1r[...])
        a1 = jax.nn.relu(
            jnp.dot(a0, w2r[...], preferred_element_type=jnp.float32)
            + b2r[...])
        z = jax.nn.relu(
            jnp.dot(a1, w3r[...], preferred_element_type=jnp.float32)
            + b3r[...])
        mask = lax.broadcasted_iota(jnp.int32, z.shape, 1) < ncls
        zm = jnp.where(mask, z, -jnp.inf)
        m = jnp.max(zm, axis=1, keepdims=True)
        ex = jnp.where(mask, jnp.exp(zm - m), 0.0)
        return ex / jnp.sum(ex, axis=1, keepdims=True), a1

    out0, _ = head(w01_ref, b01_ref, w02_ref, b02_ref, w03_ref, b03_ref, 2)
    out1, a1 = head(w11_ref, b11_ref, w12_ref, b12_ref, w13_ref, b13_ref, 5)
    out0_ref[...] = out0
    out1_ref[...] = out1
    f1_ref[...] = a1


def _heads_tc(xg, ws):
    g = xg.shape[0]
    full = lambda a: pl.BlockSpec(a.shape, lambda: (0,) * a.ndim)
    return pl.pallas_call(
        _heads_body,
        in_specs=[full(xg)] + [full(w) for w in ws],
        out_specs=[
            pl.BlockSpec((g, CW), lambda: (0, 0)),
            pl.BlockSpec((g, CW), lambda: (0, 0)),
            pl.BlockSpec((g, ws[3].shape[1]), lambda: (0, 0)),
        ],
        out_shape=[
            jax.ShapeDtypeStruct((g, CW), jnp.float32),
            jax.ShapeDtypeStruct((g, CW), jnp.float32),
            jax.ShapeDtypeStruct((g, ws[3].shape[1]), jnp.float32),
        ],
    )(xg, *ws)


def _fold_bn(w, b, bn):
    g = bn['gamma'] * lax.rsqrt(bn['rv'] + 1e-5)
    return w * g[None, :], b * g + bn['beta'] - bn['rm'] * g


def _mix_branches(h, flag, feature_s):
    def ident(_):
        return h

    def mk(which):
        def f(_):
            prob = jax.random.normal(jax.random.key(1), (h.shape[0], 1),
                                     dtype=jnp.float32)
            miu = jnp.mean(h, axis=1, keepdims=True)
            sigma = jnp.std(h, axis=1, keepdims=True)
            if which == 0:
                s = sigma * (1 + prob)
                m = miu * (1 + prob)
            else:
                miu_s = jnp.mean(feature_s, axis=1, keepdims=True)
                sigma_s = jnp.std(feature_s, axis=1, keepdims=True)
                s = prob * sigma_s + (1 - prob) * sigma
                m = prob * miu_s + (1 - prob) * miu
            return s * (h - miu) / sigma + m
        return f

    flag = jnp.asarray(flag)
    idx = jnp.where(flag == 0, 0, jnp.where(flag == 1, 1, 2))
    return lax.switch(idx, [mk(0), mk(1), ident], 0)


def kernel(x, edge_index, batch, flag, feature_s, params):
    p = params
    n, _ = x.shape
    e = edge_index.shape[1]
    h = p['conv1_w0'].shape[1]
    num_graphs = 64

    npad = _ceil_to(n, NS * 8 * LANE)
    epad = _ceil_to(e, NW * EBLK)
    epr = epad // (NW * EBLK)
    dump = n

    src = jnp.concatenate(
        [edge_index[0], jnp.zeros((epad - e,), jnp.int32)]).reshape(-1, EBLK)
    dst = jnp.concatenate(
        [edge_index[1], jnp.full((epad - e,), dump, jnp.int32)]
    ).reshape(-1, EBLK)

    w10, b1f = _fold_bn(p['conv1_w0'], p['conv1_b'], p['bn1'])
    w11, _ = _fold_bn(p['conv1_w1'], p['conv1_b'], p['bn1'])
    w20, b2f = _fold_bn(p['conv2_w0'], p['conv2_b'], p['bn2'])
    w21, _ = _fold_bn(p['conv2_w1'], p['conv2_b'], p['bn2'])
    w1cat = jnp.concatenate([w10, w11], axis=1)
    w2cat = jnp.concatenate([w20, w21], axis=1)

    degpart = _deg_sc(dst, npad, epr)
    u0, ycg, disb = _conv1_tc(x, w1cat, degpart, npad)

    zeros_rows = jnp.zeros((npad // NS, CW), jnp.float32)
    zpart1 = _prop_sc(ycg, src, dst, zeros_rows, npad, epr)
    u0b, ycg2 = _combine_matmul_tc(u0, zpart1, disb, b1f[None, :], w2cat,
                                   npad)
    zpart2 = _prop_sc(ycg2, src, dst, zeros_rows, npad, epr)
    feature = _combine_final_tc(u0b, zpart2, disb, b2f[None, :])[:n]

    feature = _mix_branches(feature, flag, feature_s)

    starts = jnp.searchsorted(
        batch, jnp.arange(num_graphs + 1, dtype=jnp.int32)).astype(jnp.int32)
    xg = _pool_tc(feature, starts, num_graphs)

    wl01, bl01 = _fold_bn(p['lin01_w'], p['lin01_b'], p['bn01'])
    wl02, bl02 = _fold_bn(p['lin02_w'], p['lin02_b'], p['bn02'])
    wl11, bl11 = _fold_bn(p['lin11_w'], p['lin11_b'], p['bn11'])
    wl12, bl12 = _fold_bn(p['lin12_w'], p['lin12_b'], p['bn12'])
    pad_w = lambda w: jnp.pad(w, ((0, 0), (0, CW - w.shape[1])))
    pad_b = lambda b: jnp.pad(b, (0, CW - b.shape[0]))
    ws = [
        wl01, bl01[None, :], wl02, bl02[None, :],
        pad_w(p['lin03_w']), pad_b(p['lin03_b'])[None, :],
        wl11, bl11[None, :], wl12, bl12[None, :],
        pad_w(p['lin13_w']), pad_b(p['lin13_b'])[None, :],
    ]
    out0p, out1p, feature1 = _heads_tc(xg, ws)
    n0 = p['lin03_w'].shape[1]
    n1 = p['lin13_w'].shape[1]
    return (out0p[:, :n0], out1p[:, :n1], feature, xg, feature1)
